# baseline (speedup 1.0000x reference)
import functools

import jax
import jax.numpy as jnp
from jax import lax
from jax.experimental import pallas as pl
from jax.experimental.pallas import tpu as pltpu
from jax.experimental.pallas import tpu_sc as plsc

_NUM_CORES = 2
_NUM_SUBCORES = 16
_LANES = 16
_NW = _NUM_CORES * _NUM_SUBCORES

_CH = 128


def _sc_call(dview, zetas, center_idx, rows, n_sh, n_ctr, feat):
    row_w = n_ctr * feat
    per_w = rows // _NW
    n_chunks = per_w // _CH

    mesh = plsc.VectorSubcoreMesh(
        core_axis_name="c", subcore_axis_name="s",
        num_cores=_NUM_CORES, num_subcores=_NUM_SUBCORES)

    @functools.partial(
        pl.kernel,
        out_type=jax.ShapeDtypeStruct((rows, n_sh), jnp.float32),
        mesh=mesh,
        scratch_types=[
            pltpu.VMEM((2, _CH, row_w), jnp.float32),
            pltpu.VMEM((2, _CH, n_sh), jnp.float32),
            pltpu.VMEM((n_sh,), jnp.float32),
            pltpu.VMEM((n_sh,), jnp.int32),
            pltpu.SemaphoreType.DMA,
            pltpu.SemaphoreType.DMA,
            pltpu.SemaphoreType.DMA,
            pltpu.SemaphoreType.DMA,
        ],
        compiler_params=pltpu.CompilerParams(needs_layout_passes=False),
    )
    def sc_kernel(d_hbm, z_hbm, ci_hbm, out_hbm, in_v, out_v, z_v, ci_v,
                  sem_i0, sem_i1, sem_o0, sem_o1):
        cid = lax.axis_index("c")
        sid = lax.axis_index("s")
        wid = sid * _NUM_CORES + cid
        base = wid * per_w

        pltpu.sync_copy(z_hbm, z_v)
        pltpu.sync_copy(ci_hbm, ci_v)

        iota = lax.iota(jnp.int32, _LANES)
        col = iota * feat + (feat - 1)
        n_grp = n_sh // _LANES
        negz = [-jnp.abs(z_v[pl.ds(_LANES * j, _LANES)]) for j in range(n_grp)]
        cidx = [ci_v[pl.ds(_LANES * j, _LANES)] for j in range(n_grp)]

        sem_in = [sem_i0, sem_i1]
        sem_out = [sem_o0, sem_o1]

        def in_copy(g, b):
            return pltpu.make_async_copy(
                d_hbm.at[pl.ds(base + g * _CH, _CH)], in_v.at[b], sem_in[b])

        def out_copy(g, b):
            return pltpu.make_async_copy(
                out_v.at[b], out_hbm.at[pl.ds(base + g * _CH, _CH)],
                sem_out[b])

        in_copy(0, 0).start()
        in_copy(1, 1).start()

        @pl.loop(0, n_chunks // 2)
        def _outer(h):
            for b in range(2):
                g = h * 2 + b
                in_copy(g, b).wait()

                @pl.when(h > 0)
                def _():
                    out_copy(g, b).wait()

                @pl.loop(0, _CH)
                def _row(r):
                    rsel = lax.broadcast(r, (_LANES,)).astype(jnp.int32)
                    r2 = plsc.load_gather(in_v.at[b], [rsel, col])
                    x = jnp.maximum(r2, jnp.float32(1e-24))
                    xi = plsc.bitcast(x, jnp.int32)
                    y = plsc.bitcast(
                        jnp.int32(0x5F3759DF) - (xi >> 1), jnp.float32)
                    h2 = x * jnp.float32(0.5)
                    y = y * (jnp.float32(1.5) - h2 * y * y)
                    y = y * (jnp.float32(1.5) - h2 * y * y)
                    rt = x * y
                    for j in range(n_grp):
                        rtg = jnp.take_along_axis(
                            rt, cidx[j], axis=0, mode="promise_in_bounds")
                        out_v[b, r, pl.ds(_LANES * j, _LANES)] = (
                            jnp.exp(negz[j] * rtg))

                out_copy(g, b).start()

                @pl.when(g + 2 < n_chunks)
                def _():
                    in_copy(g + 2, b).start()

        out_copy(n_chunks - 2, 0).wait()
        out_copy(n_chunks - 1, 1).wait()

    return sc_kernel(dview, zetas, center_idx)


@jax.jit
def kernel(diffs, zetas, center_idx):
    b, e, n_ctr, feat = diffs.shape
    n_sh = zetas.shape[0]
    rows = b * e
    dview = diffs.reshape(rows, n_ctr * feat)
    out = _sc_call(dview, zetas, center_idx.astype(jnp.int32),
                   rows, n_sh, n_ctr, feat)
    return out.reshape(b, e, n_sh)


# speedup vs baseline: 1.3851x; 1.3851x over previous
import functools

import jax
import jax.numpy as jnp
from jax import lax
from jax.experimental import pallas as pl
from jax.experimental.pallas import tpu as pltpu
from jax.experimental.pallas import tpu_sc as plsc

_NUM_CORES = 2
_NUM_SUBCORES = 16
_LANES = 16
_NW = _NUM_CORES * _NUM_SUBCORES

_CH = 128


def _sc_call(dview, zetas, center_idx, rows, n_sh, n_ctr, feat):
    row_w = n_ctr * feat
    per_w = rows // _NW
    n_chunks = per_w // _CH

    mesh = plsc.VectorSubcoreMesh(
        core_axis_name="c", subcore_axis_name="s",
        num_cores=_NUM_CORES, num_subcores=_NUM_SUBCORES)

    @functools.partial(
        pl.kernel,
        out_type=jax.ShapeDtypeStruct((rows, n_sh), jnp.float32),
        mesh=mesh,
        scratch_types=[
            pltpu.VMEM((2, _CH, row_w), jnp.float32),
            pltpu.VMEM((2, _CH, n_sh), jnp.float32),
            pltpu.VMEM((n_sh,), jnp.float32),
            pltpu.VMEM((n_sh,), jnp.int32),
            pltpu.SemaphoreType.DMA,
            pltpu.SemaphoreType.DMA,
            pltpu.SemaphoreType.DMA,
            pltpu.SemaphoreType.DMA,
        ],
        compiler_params=pltpu.CompilerParams(needs_layout_passes=False),
    )
    def sc_kernel(d_hbm, z_hbm, ci_hbm, out_hbm, in_v, out_v, z_v, ci_v,
                  sem_i0, sem_i1, sem_o0, sem_o1):
        cid = lax.axis_index("c")
        sid = lax.axis_index("s")
        wid = sid * _NUM_CORES + cid
        base = wid * per_w

        pltpu.sync_copy(z_hbm, z_v)
        pltpu.sync_copy(ci_hbm, ci_v)

        iota = lax.iota(jnp.int32, _LANES)
        col = iota * feat + (feat - 1)
        n_grp = n_sh // _LANES
        negz = [-jnp.abs(z_v[pl.ds(_LANES * j, _LANES)]) for j in range(n_grp)]
        cidx = [ci_v[pl.ds(_LANES * j, _LANES)] for j in range(n_grp)]

        sem_in = [sem_i0, sem_i1]
        sem_out = [sem_o0, sem_o1]

        def in_copy(g, b):
            return pltpu.make_async_copy(
                d_hbm.at[pl.ds(base + g * _CH, _CH)], in_v.at[b], sem_in[b])

        def out_copy(g, b):
            return pltpu.make_async_copy(
                out_v.at[b], out_hbm.at[pl.ds(base + g * _CH, _CH)],
                sem_out[b])

        in_copy(0, 0).start()
        in_copy(1, 1).start()

        @pl.loop(0, n_chunks // 2)
        def _outer(h):
            for b in range(2):
                g = h * 2 + b
                in_copy(g, b).wait()

                @pl.when(h > 0)
                def _():
                    out_copy(g, b).wait()

                @plsc.parallel_loop(0, _CH, unroll=8)
                def _row(r):
                    rsel = lax.broadcast(r, (_LANES,)).astype(jnp.int32)
                    r2 = plsc.load_gather(in_v.at[b], [rsel, col])
                    x = jnp.maximum(r2, jnp.float32(1e-24))
                    xi = plsc.bitcast(x, jnp.int32)
                    y = plsc.bitcast(
                        jnp.int32(0x5F3759DF) - (xi >> 1), jnp.float32)
                    h2 = x * jnp.float32(0.5)
                    y = y * (jnp.float32(1.5) - h2 * y * y)
                    y = y * (jnp.float32(1.5) - h2 * y * y)
                    rt = x * y
                    for j in range(n_grp):
                        rtg = jnp.take_along_axis(
                            rt, cidx[j], axis=0, mode="promise_in_bounds")
                        out_v[b, r, pl.ds(_LANES * j, _LANES)] = (
                            jnp.exp(negz[j] * rtg))

                out_copy(g, b).start()

                @pl.when(g + 2 < n_chunks)
                def _():
                    in_copy(g + 2, b).start()

        out_copy(n_chunks - 2, 0).wait()
        out_copy(n_chunks - 1, 1).wait()

    return sc_kernel(dview, zetas, center_idx)


@jax.jit
def kernel(diffs, zetas, center_idx):
    b, e, n_ctr, feat = diffs.shape
    n_sh = zetas.shape[0]
    rows = b * e
    dview = diffs.reshape(rows, n_ctr * feat)
    out = _sc_call(dview, zetas, center_idx.astype(jnp.int32),
                   rows, n_sh, n_ctr, feat)
    return out.reshape(b, e, n_sh)
